# baseline (speedup 1.0000x reference)
import functools

import jax
import jax.numpy as jnp
from jax import lax
from jax.experimental import pallas as pl
from jax.experimental.pallas import tpu as pltpu
from jax.experimental.pallas import tpu_sc as plsc

_N = 10000
_E = 320000
_F = 128
_FO = 64
_OUT = 10000
_NB = 4096
_NC = 2
_NS = 16
_CH = 128
_RPT = 80
_EPAD = _NC * _NS * _RPT * _CH
_EPT = _RPT * _CH
_NP = _N + 16
_ND = 10240
_DST = 640
_SLOTS = 4224
_SST = _SLOTS // _NS
_CBUF = _EPT + _CH
_VT = 1280

_mesh = plsc.VectorSubcoreMesh(core_axis_name="c", subcore_axis_name="s")


def _deg_body(dst2d, zeros_pad, ones_h, out, dstbuf, ones_v, deg_sh):
    c = lax.axis_index("c")
    s = lax.axis_index("s")
    pltpu.sync_copy(zeros_pad.at[pl.ds(s * _DST, _DST)],
                    deg_sh.at[pl.ds(s * _DST, _DST)])
    pltpu.sync_copy(ones_h, ones_v)
    row0 = (c * _NS + s) * _RPT
    pltpu.sync_copy(dst2d.at[pl.ds(row0, _RPT)], dstbuf)
    plsc.subcore_barrier()

    def body(j, carry):
        pltpu.sync_copy(ones_v, deg_sh.at[dstbuf.at[j]], add=True)
        return carry

    lax.fori_loop(0, _RPT, body, 0)
    plsc.subcore_barrier()
    pltpu.sync_copy(deg_sh.at[pl.ds(s * _DST, _DST)],
                    out.at[c, pl.ds(s * _DST, _DST)])


_sc_deg = functools.partial(
    pl.kernel,
    out_type=jax.ShapeDtypeStruct((_NC, _ND, _F), jnp.float32),
    mesh=_mesh,
    scratch_types=[
        pltpu.VMEM((_RPT, _CH), jnp.int32),
        pltpu.VMEM((_CH, _F), jnp.float32),
        pltpu.VMEM_SHARED((_ND, _F), jnp.float32),
    ],
)(_deg_body)


def _edge_body(src1d, dst1d, smap_h, y, zeros_nf, out,
               smap, srcv1, dstv1, csrc, cslot, crow, csrow, rows, acc_sh,
               idx_sh, gsem):
    c = lax.axis_index("c")
    s = lax.axis_index("s")
    w = c * _NS + s
    pltpu.sync_copy(zeros_nf.at[pl.ds(s * _SST, _SST)],
                    acc_sh.at[pl.ds(s * _SST, _SST)])
    pltpu.sync_copy(smap_h, smap)
    base = w * _EPT
    pltpu.sync_copy(src1d.at[pl.ds(base, _EPT)], srcv1)
    pltpu.sync_copy(dst1d.at[pl.ds(base, _EPT)], dstv1)
    plsc.subcore_barrier()

    def compact(g, cnt):
        dstv = dstv1[pl.ds(g * 16, 16)]
        srcv = srcv1[pl.ds(g * 16, 16)]
        slotv = plsc.load_gather(smap, [dstv])
        mask = slotv < _NB
        plsc.store_compressed(cslot.at[pl.ds(cnt, 16)], slotv, mask=mask)
        plsc.store_compressed(csrc.at[pl.ds(cnt, 16)], srcv, mask=mask)
        n = plsc.all_reduce_population_count(mask)
        return cnt + n[0]

    cnt = lax.fori_loop(0, _EPT // 16, compact, 0)

    def pad(i, carry):
        cslot[pl.ds(carry + i * 16, 16)] = jnp.full((16,), _NB, jnp.int32)
        csrc[pl.ds(carry + i * 16, 16)] = jnp.zeros((16,), jnp.int32)
        return carry

    lax.fori_loop(0, _CH // 16, pad, cnt)
    nrounds = (cnt + _CH - 1) // _CH

    def round_(j, carry):
        pltpu.sync_copy(csrc.at[pl.ds(j * _CH, _CH)], idx_sh.at[s, 0])
        pltpu.sync_copy(cslot.at[pl.ds(j * _CH, _CH)], idx_sh.at[s, 1])
        pltpu.sync_copy(idx_sh.at[s, 0], csrow)
        pltpu.sync_copy(idx_sh.at[s, 1], crow)
        pltpu.async_copy(y.at[csrow], rows, gsem).wait()
        pltpu.sync_copy(rows, acc_sh.at[crow], add=True)
        return carry

    lax.fori_loop(0, nrounds, round_, 0)
    plsc.subcore_barrier()
    pltpu.sync_copy(acc_sh.at[pl.ds(s * _SST, _SST)],
                    out.at[c, pl.ds(s * _SST, _SST)])


_sc_edge = functools.partial(
    pl.kernel,
    out_type=jax.ShapeDtypeStruct((_NC, _SLOTS, _F), jnp.float32),
    compiler_params=pltpu.CompilerParams(needs_layout_passes=False),
    mesh=_mesh,
    scratch_types=[
        pltpu.VMEM((_NP,), jnp.int32),
        pltpu.VMEM((_EPT,), jnp.int32),
        pltpu.VMEM((_EPT,), jnp.int32),
        pltpu.VMEM((_CBUF,), jnp.int32),
        pltpu.VMEM((_CBUF,), jnp.int32),
        pltpu.VMEM((_CH,), jnp.int32),
        pltpu.VMEM((_CH,), jnp.int32),
        pltpu.VMEM((_CH, _F), jnp.float32),
        pltpu.VMEM_SHARED((_SLOTS, _F), jnp.float32),
        pltpu.VMEM_SHARED((_NS, 2, _CH), jnp.int32),
        pltpu.SemaphoreType.DMA,
    ],
)(_edge_body)


def _batch_body(y, degp, accp, nid3d, sidx3d, ys, d0, d1, a0, a1,
                nidv, sidxv, rbuf, sem):
    c = lax.axis_index("c")
    s = lax.axis_index("s")
    w = c * _NS + s
    pltpu.sync_copy(nid3d.at[w], nidv)
    pltpu.sync_copy(sidx3d.at[w], sidxv)
    sl = pl.ds(w * 128, 128)
    pltpu.async_copy(y.at[nidv.at[0]], rbuf, sem).wait()
    pltpu.sync_copy(rbuf, ys.at[sl])
    pltpu.async_copy(degp.at[0].at[nidv.at[0]], rbuf, sem).wait()
    pltpu.sync_copy(rbuf, d0.at[sl])
    pltpu.async_copy(degp.at[1].at[nidv.at[0]], rbuf, sem).wait()
    pltpu.sync_copy(rbuf, d1.at[sl])
    pltpu.async_copy(accp.at[0].at[sidxv.at[0]], rbuf, sem).wait()
    pltpu.sync_copy(rbuf, a0.at[sl])
    pltpu.async_copy(accp.at[1].at[sidxv.at[0]], rbuf, sem).wait()
    pltpu.sync_copy(rbuf, a1.at[sl])


_sc_batch = functools.partial(
    pl.kernel,
    out_type=tuple(jax.ShapeDtypeStruct((_NB, _F), jnp.float32)
                   for _ in range(5)),
    mesh=_mesh,
    scratch_types=[
        pltpu.VMEM((1, 128), jnp.int32),
        pltpu.VMEM((1, 128), jnp.int32),
        pltpu.VMEM((128, _F), jnp.float32),
        pltpu.SemaphoreType.DMA,
    ],
)(_batch_body)


def _prep_body(x_ref, wc_ref, degp_ref, y_ref):
    deg = degp_ref[0] + degp_ref[1] + 1.0
    dinv = 1.0 / jnp.sqrt(deg)
    xw = jnp.dot(x_ref[...], wc_ref[...], preferred_element_type=jnp.float32)
    y_ref[...] = xw * dinv


_tc_prep = pl.pallas_call(
    _prep_body,
    out_shape=jax.ShapeDtypeStruct((_N, _F), jnp.float32),
)


def _hn_body(ys_ref, d0_ref, d1_ref, a0_ref, a1_ref, wlzp_ref, wlhp_ref,
             bz_ref, bh_ref, hn_ref):
    deg = d0_ref[:, 0:1] + d1_ref[:, 0:1] + 1.0
    dinv = 1.0 / jnp.sqrt(deg)
    agg = (a0_ref[...] + a1_ref[...] + ys_ref[...]) * dinv
    zl = jnp.dot(agg, wlzp_ref[...], preferred_element_type=jnp.float32)
    tl = jnp.dot(agg, wlhp_ref[...], preferred_element_type=jnp.float32)
    z = jax.nn.sigmoid(zl + bz_ref[...])
    t = jnp.tanh(tl + bh_ref[...])
    hn_ref[...] = (1.0 - z) * t


_tc_hn = pl.pallas_call(
    _hn_body,
    out_shape=jax.ShapeDtypeStruct((_NB, _FO), jnp.float32),
)


def _dec_body(zn_ref, wd1_ref, bd1_ref, wd2_ref, bd2_ref, out_ref):
    h = jax.nn.relu(
        jnp.dot(zn_ref[...], wd1_ref[...], preferred_element_type=jnp.float32)
        + bd1_ref[...])
    out_ref[...] = (
        jnp.dot(h, wd2_ref[...], preferred_element_type=jnp.float32)
        + bd2_ref[...])


_tc_dec = pl.pallas_call(
    _dec_body,
    grid=(8,),
    in_specs=[
        pl.BlockSpec((_NB, _FO), lambda j: (0, 0)),
        pl.BlockSpec((_FO, _FO), lambda j: (0, 0)),
        pl.BlockSpec((1, _FO), lambda j: (0, 0)),
        pl.BlockSpec((_FO, _VT), lambda j: (0, j)),
        pl.BlockSpec((1, _VT), lambda j: (0, j)),
    ],
    out_specs=pl.BlockSpec((_NB, _VT), lambda j: (0, j)),
    out_shape=jax.ShapeDtypeStruct((_NB, _OUT), jnp.float32),
)


def kernel(static_node_feats, edge_index, node_ids, Wz, bz, Wr, br, Wh, bh,
           Wlz, blz, Wlr, blr, Wlh, blh, Wd1, bd1, Wd2, bd2):
    x = static_node_feats
    pad = _EPAD - _E
    src1d = jnp.concatenate([edge_index[0], jnp.zeros((pad,), jnp.int32)])
    dst1d = jnp.concatenate([edge_index[1], jnp.full((pad,), _N, jnp.int32)])
    dst2d = dst1d.reshape(_EPAD // _CH, _CH)
    nid3d = node_ids.reshape(_NC * _NS, 1, 128)

    slotmap = jnp.full((_NP,), _NB, jnp.int32).at[node_ids].set(
        jnp.arange(_NB, dtype=jnp.int32))
    sidx3d = slotmap[node_ids].reshape(_NC * _NS, 1, 128)

    Wc = jnp.concatenate([Wz, Wh], axis=1)
    zpad = jnp.zeros((_FO, _FO), jnp.float32)
    Wlzp = jnp.concatenate([Wlz[:_FO], zpad])
    Wlhp = jnp.concatenate([zpad, Wlh[:_FO]])
    bz_eff = (blz + bz @ Wlz[:_FO]).reshape(1, _FO)
    bh_eff = (blh + bh @ Wlh[:_FO]).reshape(1, _FO)
    ones_ch = jnp.ones((_CH, _F), jnp.float32)
    zeros_nf = jnp.zeros((_ND, _F), jnp.float32)

    degp = _sc_deg(dst2d, zeros_nf, ones_ch)
    y = _tc_prep(x, Wc, degp[:, :_N, 0:1])
    accp = _sc_edge(src1d, dst1d, slotmap, y, zeros_nf)
    ys, d0, d1, a0, a1 = _sc_batch(y, degp, accp, nid3d, sidx3d)
    hn = _tc_hn(ys, d0, d1, a0, a1, Wlzp, Wlhp, bz_eff, bh_eff)
    logits = _tc_dec(hn, Wd1, bd1.reshape(1, _FO), Wd2,
                     bd2.reshape(1, _OUT))
    return logits


# speedup vs baseline: 1.0962x; 1.0962x over previous
import functools

import jax
import jax.numpy as jnp
from jax import lax
from jax.experimental import pallas as pl
from jax.experimental.pallas import tpu as pltpu
from jax.experimental.pallas import tpu_sc as plsc

_N = 10000
_E = 320000
_F = 128
_FO = 64
_OUT = 10000
_NB = 4096
_NC = 2
_NS = 16
_CH = 128
_RPT = 80
_EPAD = _NC * _NS * _RPT * _CH
_EPT = _RPT * _CH
_NP = _N + 16
_ND = 10240
_DST = 640
_SLOTS = 4224
_SST = _SLOTS // _NS
_CBUF = _EPT + _CH
_VT = 1280

_mesh = plsc.VectorSubcoreMesh(core_axis_name="c", subcore_axis_name="s")


def _deg_body(dst1d, zeros_pad, iota_h, out, dstv1, hist2d, iota_v, deg_sh):
    c = lax.axis_index("c")
    s = lax.axis_index("s")
    w = c * _NS + s
    pltpu.sync_copy(zeros_pad.at[pl.ds(0, _ND // 128)], hist2d)

    @pl.when(s == 0)
    def _():
        pltpu.sync_copy(zeros_pad.at[pl.ds(0, _ND // 128)], deg_sh)
    pltpu.sync_copy(iota_h, iota_v)
    pltpu.sync_copy(dst1d.at[pl.ds(w * _EPT, _EPT)], dstv1)

    def grp(g, carry):
        v = dstv1[pl.ds(g * 16, 16)]
        rows = jax.lax.shift_right_logical(v, 7)
        cols = jax.lax.bitwise_and(v, 127)
        plsc.addupdate_scatter(hist2d, [rows, cols],
                               jnp.ones((16,), jnp.float32))
        return carry

    lax.fori_loop(0, _EPT // 16, grp, 0)
    plsc.subcore_barrier()
    pltpu.sync_copy(hist2d, deg_sh.at[iota_v], add=True)
    plsc.subcore_barrier()

    @pl.when(s < 5)
    def _():
        pltpu.sync_copy(deg_sh.at[pl.ds(s * 16, 16)],
                        out.at[c, pl.ds(s * 16, 16)])


_sc_deg = functools.partial(
    pl.kernel,
    out_type=jax.ShapeDtypeStruct((_NC, _ND // 128, _F), jnp.float32),
    compiler_params=pltpu.CompilerParams(needs_layout_passes=False),
    mesh=_mesh,
    scratch_types=[
        pltpu.VMEM((_EPT,), jnp.int32),
        pltpu.VMEM((_ND // 128, _F), jnp.float32),
        pltpu.VMEM((_ND // 128,), jnp.int32),
        pltpu.VMEM_SHARED((_ND // 128, _F), jnp.float32),
    ],
)(_deg_body)


def _edge_body(src1d, dst1d, smap_h, y, zeros_nf, out,
               smap, srcv1, dstv1, csrc, cslot, crow, csrow, rows, acc_sh,
               idx_sh, gsem):
    c = lax.axis_index("c")
    s = lax.axis_index("s")
    w = c * _NS + s
    pltpu.sync_copy(zeros_nf.at[pl.ds(s * _SST, _SST)],
                    acc_sh.at[pl.ds(s * _SST, _SST)])
    pltpu.sync_copy(smap_h, smap)
    base = w * _EPT
    pltpu.sync_copy(src1d.at[pl.ds(base, _EPT)], srcv1)
    pltpu.sync_copy(dst1d.at[pl.ds(base, _EPT)], dstv1)
    plsc.subcore_barrier()

    def compact(g, cnt):
        dstv = dstv1[pl.ds(g * 16, 16)]
        srcv = srcv1[pl.ds(g * 16, 16)]
        slotv = plsc.load_gather(smap, [dstv])
        mask = slotv < _NB
        plsc.store_compressed(cslot.at[pl.ds(cnt, 16)], slotv, mask=mask)
        plsc.store_compressed(csrc.at[pl.ds(cnt, 16)], srcv, mask=mask)
        n = plsc.all_reduce_population_count(mask)
        return cnt + n[0]

    cnt = lax.fori_loop(0, _EPT // 16, compact, 0)

    def pad(i, carry):
        cslot[pl.ds(carry + i * 16, 16)] = jnp.full((16,), _NB, jnp.int32)
        csrc[pl.ds(carry + i * 16, 16)] = jnp.zeros((16,), jnp.int32)
        return carry

    lax.fori_loop(0, _CH // 16, pad, cnt)
    nrounds = (cnt + _CH - 1) // _CH

    def round_(j, carry):
        pltpu.sync_copy(csrc.at[pl.ds(j * _CH, _CH)], idx_sh.at[s, 0])
        pltpu.sync_copy(cslot.at[pl.ds(j * _CH, _CH)], idx_sh.at[s, 1])
        pltpu.sync_copy(idx_sh.at[s, 0], csrow)
        pltpu.sync_copy(idx_sh.at[s, 1], crow)
        pltpu.async_copy(y.at[csrow], rows, gsem).wait()
        pltpu.sync_copy(rows, acc_sh.at[crow], add=True)
        return carry

    lax.fori_loop(0, nrounds, round_, 0)
    plsc.subcore_barrier()
    pltpu.sync_copy(acc_sh.at[pl.ds(s * _SST, _SST)],
                    out.at[c, pl.ds(s * _SST, _SST)])


_sc_edge = functools.partial(
    pl.kernel,
    out_type=jax.ShapeDtypeStruct((_NC, _SLOTS, _F), jnp.float32),
    compiler_params=pltpu.CompilerParams(needs_layout_passes=False),
    mesh=_mesh,
    scratch_types=[
        pltpu.VMEM((_NP,), jnp.int32),
        pltpu.VMEM((_EPT,), jnp.int32),
        pltpu.VMEM((_EPT,), jnp.int32),
        pltpu.VMEM((_CBUF,), jnp.int32),
        pltpu.VMEM((_CBUF,), jnp.int32),
        pltpu.VMEM((_CH,), jnp.int32),
        pltpu.VMEM((_CH,), jnp.int32),
        pltpu.VMEM((_CH, _F), jnp.float32),
        pltpu.VMEM_SHARED((_SLOTS, _F), jnp.float32),
        pltpu.VMEM_SHARED((_NS, 2, _CH), jnp.int32),
        pltpu.SemaphoreType.DMA,
    ],
)(_edge_body)


def _batch_body(y, degb, accp, nid3d, sidx3d, ys, dg, a0, a1,
                nidv, sidxv, rbuf, sem):
    c = lax.axis_index("c")
    s = lax.axis_index("s")
    w = c * _NS + s
    pltpu.sync_copy(nid3d.at[w], nidv)
    pltpu.sync_copy(sidx3d.at[w], sidxv)
    sl = pl.ds(w * 128, 128)
    pltpu.async_copy(y.at[nidv.at[0]], rbuf, sem).wait()
    pltpu.sync_copy(rbuf, ys.at[sl])
    pltpu.async_copy(degb.at[nidv.at[0]], rbuf, sem).wait()
    pltpu.sync_copy(rbuf, dg.at[sl])
    pltpu.async_copy(accp.at[0].at[sidxv.at[0]], rbuf, sem).wait()
    pltpu.sync_copy(rbuf, a0.at[sl])
    pltpu.async_copy(accp.at[1].at[sidxv.at[0]], rbuf, sem).wait()
    pltpu.sync_copy(rbuf, a1.at[sl])


_sc_batch = functools.partial(
    pl.kernel,
    out_type=tuple(jax.ShapeDtypeStruct((_NB, _F), jnp.float32)
                   for _ in range(4)),
    mesh=_mesh,
    scratch_types=[
        pltpu.VMEM((1, 128), jnp.int32),
        pltpu.VMEM((1, 128), jnp.int32),
        pltpu.VMEM((128, _F), jnp.float32),
        pltpu.SemaphoreType.DMA,
    ],
)(_batch_body)


def _prep_body(x_ref, wc_ref, degp_ref, y_ref, degb_ref):
    deg = degp_ref[0] + degp_ref[1] + 1.0
    dinv = 1.0 / jnp.sqrt(deg)
    xw = jnp.dot(x_ref[...], wc_ref[...], preferred_element_type=jnp.float32)
    y_ref[...] = xw * dinv
    degb_ref[...] = jnp.broadcast_to(deg, (_N, _F))


_tc_prep = pl.pallas_call(
    _prep_body,
    out_shape=(
        jax.ShapeDtypeStruct((_N, _F), jnp.float32),
        jax.ShapeDtypeStruct((_N, _F), jnp.float32),
    ),
)


def _hn_body(ys_ref, dg_ref, a0_ref, a1_ref, wlzp_ref, wlhp_ref,
             bz_ref, bh_ref, hn_ref):
    dinv = 1.0 / jnp.sqrt(dg_ref[:, 0:1])
    agg = (a0_ref[...] + a1_ref[...] + ys_ref[...]) * dinv
    zl = jnp.dot(agg, wlzp_ref[...], preferred_element_type=jnp.float32)
    tl = jnp.dot(agg, wlhp_ref[...], preferred_element_type=jnp.float32)
    z = jax.nn.sigmoid(zl + bz_ref[...])
    t = jnp.tanh(tl + bh_ref[...])
    hn_ref[...] = (1.0 - z) * t


_tc_hn = pl.pallas_call(
    _hn_body,
    out_shape=jax.ShapeDtypeStruct((_NB, _FO), jnp.float32),
)


def _dec_body(zn_ref, wd1_ref, bd1_ref, wd2_ref, bd2_ref, out_ref):
    h = jax.nn.relu(
        jnp.dot(zn_ref[...], wd1_ref[...], preferred_element_type=jnp.float32)
        + bd1_ref[...])
    out_ref[...] = (
        jnp.dot(h, wd2_ref[...], preferred_element_type=jnp.float32)
        + bd2_ref[...])


_tc_dec = pl.pallas_call(
    _dec_body,
    grid=(8,),
    in_specs=[
        pl.BlockSpec((_NB, _FO), lambda j: (0, 0)),
        pl.BlockSpec((_FO, _FO), lambda j: (0, 0)),
        pl.BlockSpec((1, _FO), lambda j: (0, 0)),
        pl.BlockSpec((_FO, _VT), lambda j: (0, j)),
        pl.BlockSpec((1, _VT), lambda j: (0, j)),
    ],
    out_specs=pl.BlockSpec((_NB, _VT), lambda j: (0, j)),
    out_shape=jax.ShapeDtypeStruct((_NB, _OUT), jnp.float32),
)


def kernel(static_node_feats, edge_index, node_ids, Wz, bz, Wr, br, Wh, bh,
           Wlz, blz, Wlr, blr, Wlh, blh, Wd1, bd1, Wd2, bd2):
    x = static_node_feats
    pad = _EPAD - _E
    src1d = jnp.concatenate([edge_index[0], jnp.zeros((pad,), jnp.int32)])
    dst1d = jnp.concatenate([edge_index[1], jnp.full((pad,), _N, jnp.int32)])
    nid3d = node_ids.reshape(_NC * _NS, 1, 128)

    slotmap = jnp.full((_NP,), _NB, jnp.int32).at[node_ids].set(
        jnp.arange(_NB, dtype=jnp.int32))
    sidx3d = slotmap[node_ids].reshape(_NC * _NS, 1, 128)

    Wc = jnp.concatenate([Wz, Wh], axis=1)
    zpad = jnp.zeros((_FO, _FO), jnp.float32)
    Wlzp = jnp.concatenate([Wlz[:_FO], zpad])
    Wlhp = jnp.concatenate([zpad, Wlh[:_FO]])
    bz_eff = (blz + bz @ Wlz[:_FO]).reshape(1, _FO)
    bh_eff = (blh + bh @ Wlh[:_FO]).reshape(1, _FO)
    iota80 = jnp.arange(_ND // 128, dtype=jnp.int32)
    zeros_nf = jnp.zeros((_ND, _F), jnp.float32)

    degp = _sc_deg(dst1d, zeros_nf, iota80)
    degc = degp.reshape(_NC, _ND, 1)[:, :_N]
    y, degb = _tc_prep(x, Wc, degc)
    accp = _sc_edge(src1d, dst1d, slotmap, y, zeros_nf)
    ys, dg, a0, a1 = _sc_batch(y, degb, accp, nid3d, sidx3d)
    hn = _tc_hn(ys, dg, a0, a1, Wlzp, Wlhp, bz_eff, bh_eff)
    logits = _tc_dec(hn, Wd1, bd1.reshape(1, _FO), Wd2,
                     bd2.reshape(1, _OUT))
    return logits


# speedup vs baseline: 1.1411x; 1.0410x over previous
import functools

import jax
import jax.numpy as jnp
from jax import lax
from jax.experimental import pallas as pl
from jax.experimental.pallas import tpu as pltpu
from jax.experimental.pallas import tpu_sc as plsc

_N = 10000
_E = 320000
_F = 128
_FO = 64
_OUT = 10000
_NB = 4096
_NC = 2
_NS = 16
_CH = 128
_RPT = 80
_EPAD = _NC * _NS * _RPT * _CH
_EPT = _RPT * _CH
_NP = _N + 16
_ND = 10240
_DST = 640
_SLOTS = 4224
_SST = _SLOTS // _NS
_CBUF = _EPT + _CH
_VT = 1280

_mesh = plsc.VectorSubcoreMesh(core_axis_name="c", subcore_axis_name="s")


def _deg_body(dst1d, zeros_pad, iota_h, out, dstv1, hist2d, iota_v, deg_sh):
    c = lax.axis_index("c")
    s = lax.axis_index("s")
    w = c * _NS + s
    pltpu.sync_copy(zeros_pad.at[pl.ds(0, _ND // 128)], hist2d)

    @pl.when(s == 0)
    def _():
        pltpu.sync_copy(zeros_pad.at[pl.ds(0, _ND // 128)], deg_sh)
    pltpu.sync_copy(iota_h, iota_v)
    pltpu.sync_copy(dst1d.at[pl.ds(w * _EPT, _EPT)], dstv1)

    def grp(g, carry):
        v = dstv1[pl.ds(g * 16, 16)]
        rows = jax.lax.shift_right_logical(v, 7)
        cols = jax.lax.bitwise_and(v, 127)
        plsc.addupdate_scatter(hist2d, [rows, cols],
                               jnp.ones((16,), jnp.float32))
        return carry

    lax.fori_loop(0, _EPT // 16, grp, 0)
    plsc.subcore_barrier()
    pltpu.sync_copy(hist2d, deg_sh.at[iota_v], add=True)
    plsc.subcore_barrier()

    @pl.when(s < 5)
    def _():
        pltpu.sync_copy(deg_sh.at[pl.ds(s * 16, 16)],
                        out.at[c, pl.ds(s * 16, 16)])


_sc_deg = functools.partial(
    pl.kernel,
    out_type=jax.ShapeDtypeStruct((_NC, _ND // 128, _F), jnp.float32),
    compiler_params=pltpu.CompilerParams(needs_layout_passes=False),
    mesh=_mesh,
    scratch_types=[
        pltpu.VMEM((_EPT,), jnp.int32),
        pltpu.VMEM((_ND // 128, _F), jnp.float32),
        pltpu.VMEM((_ND // 128,), jnp.int32),
        pltpu.VMEM_SHARED((_ND // 128, _F), jnp.float32),
    ],
)(_deg_body)


def _edge_body(src1d, dst1d, smap_h, y, zeros_nf, out,
               smap, srcv1, dstv1, csrc, cslot, cr0, cr1, cs0, cs1, rw0, rw1,
               acc_sh, idx_sh, g0, g1, s0, s1):
    crow = [cr0, cr1]
    csrow = [cs0, cs1]
    rows = [rw0, rw1]
    gsems = [g0, g1]
    ssems = [s0, s1]
    c = lax.axis_index("c")
    s = lax.axis_index("s")
    w = c * _NS + s
    pltpu.sync_copy(zeros_nf.at[pl.ds(s * _SST, _SST)],
                    acc_sh.at[pl.ds(s * _SST, _SST)])
    pltpu.sync_copy(smap_h, smap)
    base = w * _EPT
    pltpu.sync_copy(src1d.at[pl.ds(base, _EPT)], srcv1)
    pltpu.sync_copy(dst1d.at[pl.ds(base, _EPT)], dstv1)
    plsc.subcore_barrier()

    def compact(g, cnt):
        dstv = dstv1[pl.ds(g * 16, 16)]
        srcv = srcv1[pl.ds(g * 16, 16)]
        slotv = plsc.load_gather(smap, [dstv])
        mask = slotv < _NB
        plsc.store_compressed(cslot.at[pl.ds(cnt, 16)], slotv, mask=mask)
        plsc.store_compressed(csrc.at[pl.ds(cnt, 16)], srcv, mask=mask)
        n = plsc.all_reduce_population_count(mask)
        return cnt + n[0]

    cnt = lax.fori_loop(0, _EPT // 16, compact, 0)

    def pad(i, carry):
        cslot[pl.ds(carry + i * 16, 16)] = jnp.full((16,), _NB, jnp.int32)
        csrc[pl.ds(carry + i * 16, 16)] = jnp.zeros((16,), jnp.int32)
        return carry

    lax.fori_loop(0, _CH // 16, pad, cnt)
    nrounds = (cnt + _CH - 1) // _CH

    def round_(jj, carry):
        for b in range(2):
            j = jj * 2 + b

            @pl.when(j < nrounds)
            def _(b=b, j=j, jj=jj):
                @pl.when(jj > 0)
                def _():
                    pltpu.make_async_copy(rows[b], acc_sh.at[crow[b]],
                                          ssems[b]).wait()
                pltpu.sync_copy(csrc.at[pl.ds(j * _CH, _CH)],
                                idx_sh.at[s, 2 * b])
                pltpu.sync_copy(cslot.at[pl.ds(j * _CH, _CH)],
                                idx_sh.at[s, 2 * b + 1])
                pltpu.sync_copy(idx_sh.at[s, 2 * b], csrow[b])
                pltpu.sync_copy(idx_sh.at[s, 2 * b + 1], crow[b])
                pltpu.async_copy(y.at[csrow[b]], rows[b], gsems[b])
        for b in range(2):
            j = jj * 2 + b

            @pl.when(j < nrounds)
            def _(b=b, j=j):
                pltpu.make_async_copy(y.at[csrow[b]], rows[b],
                                      gsems[b]).wait()
                pltpu.async_copy(rows[b], acc_sh.at[crow[b]], ssems[b],
                                 add=True)
        return carry

    lax.fori_loop(0, (nrounds + 1) // 2, round_, 0)
    for b in range(2):
        @pl.when(nrounds > b)
        def _(b=b):
            pltpu.make_async_copy(rows[b], acc_sh.at[crow[b]],
                                  ssems[b]).wait()
    plsc.subcore_barrier()
    pltpu.sync_copy(acc_sh.at[pl.ds(s * _SST, _SST)],
                    out.at[c, pl.ds(s * _SST, _SST)])


_sc_edge = functools.partial(
    pl.kernel,
    out_type=jax.ShapeDtypeStruct((_NC, _SLOTS, _F), jnp.float32),
    compiler_params=pltpu.CompilerParams(needs_layout_passes=False),
    mesh=_mesh,
    scratch_types=[
        pltpu.VMEM((_NP,), jnp.int32),
        pltpu.VMEM((_EPT,), jnp.int32),
        pltpu.VMEM((_EPT,), jnp.int32),
        pltpu.VMEM((_CBUF,), jnp.int32),
        pltpu.VMEM((_CBUF,), jnp.int32),
        pltpu.VMEM((_CH,), jnp.int32),
        pltpu.VMEM((_CH,), jnp.int32),
        pltpu.VMEM((_CH,), jnp.int32),
        pltpu.VMEM((_CH,), jnp.int32),
        pltpu.VMEM((_CH, _F), jnp.float32),
        pltpu.VMEM((_CH, _F), jnp.float32),
        pltpu.VMEM_SHARED((_SLOTS, _F), jnp.float32),
        pltpu.VMEM_SHARED((_NS, 4, _CH), jnp.int32),
        pltpu.SemaphoreType.DMA,
        pltpu.SemaphoreType.DMA,
        pltpu.SemaphoreType.DMA,
        pltpu.SemaphoreType.DMA,
    ],
)(_edge_body)


def _batch_body(y, degb, accp, nid3d, sidx3d, ys, dg, a0, a1,
                nidv, sidxv, rbuf, sem):
    c = lax.axis_index("c")
    s = lax.axis_index("s")
    w = c * _NS + s
    pltpu.sync_copy(nid3d.at[w], nidv)
    pltpu.sync_copy(sidx3d.at[w], sidxv)
    sl = pl.ds(w * 128, 128)
    pltpu.async_copy(y.at[nidv.at[0]], rbuf, sem).wait()
    pltpu.sync_copy(rbuf, ys.at[sl])
    pltpu.async_copy(degb.at[nidv.at[0]], rbuf, sem).wait()
    pltpu.sync_copy(rbuf, dg.at[sl])
    pltpu.async_copy(accp.at[0].at[sidxv.at[0]], rbuf, sem).wait()
    pltpu.sync_copy(rbuf, a0.at[sl])
    pltpu.async_copy(accp.at[1].at[sidxv.at[0]], rbuf, sem).wait()
    pltpu.sync_copy(rbuf, a1.at[sl])


_sc_batch = functools.partial(
    pl.kernel,
    out_type=tuple(jax.ShapeDtypeStruct((_NB, _F), jnp.float32)
                   for _ in range(4)),
    mesh=_mesh,
    scratch_types=[
        pltpu.VMEM((1, 128), jnp.int32),
        pltpu.VMEM((1, 128), jnp.int32),
        pltpu.VMEM((128, _F), jnp.float32),
        pltpu.SemaphoreType.DMA,
    ],
)(_batch_body)


def _prep_body(x_ref, wc_ref, degp_ref, y_ref, degb_ref):
    deg = degp_ref[0] + degp_ref[1] + 1.0
    dinv = 1.0 / jnp.sqrt(deg)
    xw = jnp.dot(x_ref[...], wc_ref[...], preferred_element_type=jnp.float32)
    y_ref[...] = xw * dinv
    degb_ref[...] = jnp.broadcast_to(deg, (_N, _F))


_tc_prep = pl.pallas_call(
    _prep_body,
    out_shape=(
        jax.ShapeDtypeStruct((_N, _F), jnp.float32),
        jax.ShapeDtypeStruct((_N, _F), jnp.float32),
    ),
)


def _hn_body(ys_ref, dg_ref, a0_ref, a1_ref, wlzp_ref, wlhp_ref,
             bz_ref, bh_ref, hn_ref):
    dinv = 1.0 / jnp.sqrt(dg_ref[:, 0:1])
    agg = (a0_ref[...] + a1_ref[...] + ys_ref[...]) * dinv
    zl = jnp.dot(agg, wlzp_ref[...], preferred_element_type=jnp.float32)
    tl = jnp.dot(agg, wlhp_ref[...], preferred_element_type=jnp.float32)
    z = jax.nn.sigmoid(zl + bz_ref[...])
    t = jnp.tanh(tl + bh_ref[...])
    hn_ref[...] = (1.0 - z) * t


_tc_hn = pl.pallas_call(
    _hn_body,
    out_shape=jax.ShapeDtypeStruct((_NB, _FO), jnp.float32),
)


def _dec_body(zn_ref, wd1_ref, bd1_ref, wd2_ref, bd2_ref, out_ref):
    h = jax.nn.relu(
        jnp.dot(zn_ref[...], wd1_ref[...], preferred_element_type=jnp.float32)
        + bd1_ref[...])
    out_ref[...] = (
        jnp.dot(h, wd2_ref[...], preferred_element_type=jnp.float32)
        + bd2_ref[...])


_tc_dec = pl.pallas_call(
    _dec_body,
    grid=(8,),
    in_specs=[
        pl.BlockSpec((_NB, _FO), lambda j: (0, 0)),
        pl.BlockSpec((_FO, _FO), lambda j: (0, 0)),
        pl.BlockSpec((1, _FO), lambda j: (0, 0)),
        pl.BlockSpec((_FO, _VT), lambda j: (0, j)),
        pl.BlockSpec((1, _VT), lambda j: (0, j)),
    ],
    out_specs=pl.BlockSpec((_NB, _VT), lambda j: (0, j)),
    out_shape=jax.ShapeDtypeStruct((_NB, _OUT), jnp.float32),
)


def kernel(static_node_feats, edge_index, node_ids, Wz, bz, Wr, br, Wh, bh,
           Wlz, blz, Wlr, blr, Wlh, blh, Wd1, bd1, Wd2, bd2):
    x = static_node_feats
    pad = _EPAD - _E
    src1d = jnp.concatenate([edge_index[0], jnp.zeros((pad,), jnp.int32)])
    dst1d = jnp.concatenate([edge_index[1], jnp.full((pad,), _N, jnp.int32)])
    nid3d = node_ids.reshape(_NC * _NS, 1, 128)

    slotmap = jnp.full((_NP,), _NB, jnp.int32).at[node_ids].set(
        jnp.arange(_NB, dtype=jnp.int32))
    sidx3d = slotmap[node_ids].reshape(_NC * _NS, 1, 128)

    Wc = jnp.concatenate([Wz, Wh], axis=1)
    zpad = jnp.zeros((_FO, _FO), jnp.float32)
    Wlzp = jnp.concatenate([Wlz[:_FO], zpad])
    Wlhp = jnp.concatenate([zpad, Wlh[:_FO]])
    bz_eff = (blz + bz @ Wlz[:_FO]).reshape(1, _FO)
    bh_eff = (blh + bh @ Wlh[:_FO]).reshape(1, _FO)
    iota80 = jnp.arange(_ND // 128, dtype=jnp.int32)
    zeros_nf = jnp.zeros((_ND, _F), jnp.float32)

    degp = _sc_deg(dst1d, zeros_nf, iota80)
    degc = degp.reshape(_NC, _ND, 1)[:, :_N]
    y, degb = _tc_prep(x, Wc, degc)
    accp = _sc_edge(src1d, dst1d, slotmap, y, zeros_nf)
    ys, dg, a0, a1 = _sc_batch(y, degb, accp, nid3d, sidx3d)
    hn = _tc_hn(ys, dg, a0, a1, Wlzp, Wlhp, bz_eff, bh_eff)
    logits = _tc_dec(hn, Wd1, bd1.reshape(1, _FO), Wd2,
                     bd2.reshape(1, _OUT))
    return logits
